# baseline (speedup 1.0000x reference)
import functools

import jax
import jax.numpy as jnp
from jax import lax
from jax.experimental import pallas as pl
from jax.experimental.pallas import tpu as pltpu
from jax.experimental.pallas import tpu_sc as plsc

_SCALE = 10.0
_D = 32
_C = 128


@functools.lru_cache(maxsize=None)
def _build(B, V):
    info = plsc.get_sparse_core_info()
    NW = info.num_cores * info.num_subcores
    W = B // NW
    G = W // _C
    mesh = plsc.VectorSubcoreMesh(core_axis_name="c", subcore_axis_name="s")

    @functools.partial(
        pl.kernel,
        mesh=mesh,
        out_type=jax.ShapeDtypeStruct((B, _D), jnp.float32),
        scratch_types=[
            pltpu.VMEM((G, _C), jnp.int32),
            pltpu.VMEM((_C, _D), jnp.float32),
            pltpu.SemaphoreType.DMA,
        ],
        compiler_params=pltpu.CompilerParams(use_tc_tiling_on_sc=False),
    )
    def k(idx_hbm, table_hbm, out_hbm, idx_v, rows_v, sem):
        cid = lax.axis_index("c")
        sid = lax.axis_index("s")
        wid = sid * info.num_cores + cid
        pltpu.sync_copy(idx_hbm.at[wid], idx_v)

        def chunk(g, _):
            pltpu.async_copy(table_hbm.at[idx_v.at[g]], rows_v, sem).wait()

            def scale_row(r, _):
                rows_v[r, pl.ds(0, 16)] = rows_v[r, pl.ds(0, 16)] * _SCALE
                rows_v[r, pl.ds(16, 16)] = rows_v[r, pl.ds(16, 16)] * _SCALE
                return 0

            lax.fori_loop(0, _C, scale_row, 0)
            pltpu.sync_copy(rows_v, out_hbm.at[pl.ds((wid * G + g) * _C, _C)])
            return 0

        lax.fori_loop(0, G, chunk, 0)

    return k


@jax.jit
def kernel(x, weight):
    B = x.size
    info_nw = 32
    idx = x.reshape(info_nw, B // info_nw // _C, _C).astype(jnp.int32)
    out = _build(B, weight.shape[0])(idx, weight)
    return out.reshape(x.shape + (weight.shape[1],))


# speedup vs baseline: 1.1531x; 1.1531x over previous
import functools

import jax
import jax.numpy as jnp
from jax import lax
from jax.experimental import pallas as pl
from jax.experimental.pallas import tpu as pltpu
from jax.experimental.pallas import tpu_sc as plsc

_SCALE = 10.0
_D = 32
_C = 128
_K = 5
_NBUF = 4


@functools.lru_cache(maxsize=None)
def _build(B, V):
    info = plsc.get_sparse_core_info()
    NW = info.num_cores * info.num_subcores
    W = B // NW
    G = W // _C
    R = _K * _C
    NG = W // R
    assert NG * R == W and G * _C == W
    mesh = plsc.VectorSubcoreMesh(core_axis_name="c", subcore_axis_name="s")

    @functools.partial(
        pl.kernel,
        mesh=mesh,
        out_type=jax.ShapeDtypeStruct((B, _D), jnp.float32),
        scratch_types=[
            pltpu.VMEM((G, _C), jnp.int32),
            pltpu.VMEM((_NBUF, R, _D), jnp.float32),
            [pltpu.SemaphoreType.DMA] * _NBUF,
            [pltpu.SemaphoreType.DMA] * _NBUF,
        ],
        compiler_params=pltpu.CompilerParams(use_tc_tiling_on_sc=False),
    )
    def k(idx_hbm, table_hbm, out_hbm, idx_v, rows_v, gsems, ssems):
        cid = lax.axis_index("c")
        sid = lax.axis_index("s")
        wid = sid * info.num_cores + cid
        pltpu.sync_copy(idx_hbm.at[wid], idx_v)

        def fire_gather(g):
            s = g % _NBUF
            return [
                pltpu.async_copy(
                    table_hbm.at[idx_v.at[g * _K + j]],
                    rows_v.at[s, pl.ds(j * _C, _C)],
                    gsems[s],
                )
                for j in range(_K)
            ]

        def fire_store(g):
            s = g % _NBUF
            return pltpu.async_copy(
                rows_v.at[s],
                out_hbm.at[pl.ds(wid * W + g * R, R)],
                ssems[s],
            )

        def scale(g):
            s = g % _NBUF

            @plsc.parallel_loop(0, R, 1, unroll=8)
            def _(r):
                rows_v[s, r, pl.ds(0, 16)] = rows_v[s, r, pl.ds(0, 16)] * _SCALE
                rows_v[s, r, pl.ds(16, 16)] = rows_v[s, r, pl.ds(16, 16)] * _SCALE

        gh = [None] * NG
        sh = [None] * NG
        gh[0] = fire_gather(0)
        for g in range(NG):
            if g + 1 < NG:
                if g + 1 >= _NBUF:
                    sh[g + 1 - _NBUF].wait()
                gh[g + 1] = fire_gather(g + 1)
            for h in gh[g]:
                h.wait()
            scale(g)
            sh[g] = fire_store(g)
        for g in range(NG - _NBUF, NG):
            sh[g].wait()

    return k


@jax.jit
def kernel(x, weight):
    B = x.size
    nw = 32
    idx = x.reshape(nw, B // nw // _C, _C).astype(jnp.int32)
    out = _build(B, weight.shape[0])(idx, weight)
    return out.reshape(x.shape + (weight.shape[1],))


# speedup vs baseline: 1.8717x; 1.6232x over previous
import functools

import jax
import jax.numpy as jnp
from jax import lax
from jax.experimental import pallas as pl
from jax.experimental.pallas import tpu as pltpu
from jax.experimental.pallas import tpu_sc as plsc

_SCALE = 10.0
_D = 32
_KX = 8
_NBUF = 4


@functools.lru_cache(maxsize=None)
def _build(NB, S, V):
    info = plsc.get_sparse_core_info()
    NW = info.num_cores * info.num_subcores
    WX = NB // NW
    NG = WX // _KX
    R = _KX * S
    assert NG * _KX == WX and NG % _NBUF == 0
    mesh = plsc.VectorSubcoreMesh(core_axis_name="c", subcore_axis_name="s")

    @functools.partial(
        pl.kernel,
        mesh=mesh,
        out_type=jax.ShapeDtypeStruct((NB, S, _D), jnp.float32),
        scratch_types=[
            pltpu.VMEM((WX, S), jnp.int32),
            pltpu.VMEM((_NBUF, R, _D), jnp.float32),
            [pltpu.SemaphoreType.DMA] * _NBUF,
            [pltpu.SemaphoreType.DMA] * _NBUF,
        ],
        compiler_params=pltpu.CompilerParams(use_tc_tiling_on_sc=False),
    )
    def k(x_hbm, table_hbm, out_hbm, idx_v, rows_v, gsems, ssems):
        cid = lax.axis_index("c")
        sid = lax.axis_index("s")
        wid = sid * info.num_cores + cid
        xbase = wid * WX
        pltpu.sync_copy(x_hbm.at[pl.ds(xbase, WX)], idx_v)

        def fire_gathers(g, s):
            for j in range(_KX):
                pltpu.async_copy(
                    table_hbm.at[idx_v.at[g * _KX + j]],
                    rows_v.at[s, pl.ds(j * S, S)],
                    gsems[s],
                )

        def wait_gathers(s):
            for j in range(_KX):
                pltpu.make_async_copy(
                    table_hbm.at[idx_v.at[0]],
                    rows_v.at[s, pl.ds(j * S, S)],
                    gsems[s],
                ).wait()

        def fire_stores(g, s):
            for j in range(_KX):
                pltpu.async_copy(
                    rows_v.at[s, pl.ds(j * S, S)],
                    out_hbm.at[xbase + g * _KX + j],
                    ssems[s],
                )

        def wait_stores(s):
            for j in range(_KX):
                pltpu.make_async_copy(
                    rows_v.at[s, pl.ds(j * S, S)],
                    out_hbm.at[0],
                    ssems[s],
                ).wait()

        def scale(s):
            @plsc.parallel_loop(0, R, 1, unroll=8)
            def _(r):
                rows_v[s, r, pl.ds(0, 16)] = rows_v[s, r, pl.ds(0, 16)] * _SCALE
                rows_v[s, r, pl.ds(16, 16)] = rows_v[s, r, pl.ds(16, 16)] * _SCALE

        for b in range(_NBUF - 1):
            fire_gathers(b, b)

        def outer(g2, _):
            for ph in range(_NBUF):
                g = g2 * _NBUF + ph
                sp = (ph + _NBUF - 1) % _NBUF
                def do_drain():
                    wait_stores(sp)

                def do_refill():
                    fire_gathers(g + _NBUF - 1, sp)

                if ph == 0:

                    @pl.when(g2 >= 1)
                    def _():
                        do_drain()

                else:
                    do_drain()

                @pl.when(g + _NBUF - 1 < NG)
                def _():
                    do_refill()

                wait_gathers(ph)
                scale(ph)
                fire_stores(g, ph)
            return 0

        lax.fori_loop(0, NG // _NBUF, outer, 0)

        wait_stores((NG - 1) % _NBUF)

    return k


@jax.jit
def kernel(x, weight):
    out = _build(x.shape[0], x.shape[1], weight.shape[0])(
        x.astype(jnp.int32), weight
    )
    return out


# speedup vs baseline: 2.0722x; 1.1071x over previous
import functools

import jax
import jax.numpy as jnp
from jax import lax
from jax.experimental import pallas as pl
from jax.experimental.pallas import tpu as pltpu
from jax.experimental.pallas import tpu_sc as plsc

_SCALE = 10.0
_D = 32
_BB = 128
_NSLOT = 4
_SP = 64


@functools.lru_cache(maxsize=None)
def _build(NB, S, V):
    info = plsc.get_sparse_core_info()
    NW = info.num_cores * info.num_subcores
    NBB = NB // _BB
    BPW = NBB // NW
    WX = NB // NW
    assert BPW == _NSLOT
    mesh = plsc.VectorSubcoreMesh(core_axis_name="c", subcore_axis_name="s")

    @functools.partial(
        pl.kernel,
        mesh=mesh,
        out_type=jax.ShapeDtypeStruct((S, _D // 8, NBB, 8, 128), jnp.float32),
        scratch_types=[
            pltpu.VMEM((WX, _SP), jnp.int32),
            pltpu.VMEM((_SP, WX), jnp.int32),
            pltpu.VMEM((_NSLOT, _BB, _D), jnp.float32),
            pltpu.VMEM((_NSLOT, _D // 8, 8, 128), jnp.float32),
            [pltpu.SemaphoreType.DMA] * _NSLOT,
            [pltpu.SemaphoreType.DMA] * _NSLOT,
        ],
        compiler_params=pltpu.CompilerParams(
            use_tc_tiling_on_sc=False, needs_layout_passes=False
        ),
    )
    def k(x_hbm, table_hbm, out_hbm, xv, cv, rows_v, tiles_v, gsems, ssems):
        cid = lax.axis_index("c")
        sid = lax.axis_index("s")
        wid = sid * info.num_cores + cid
        xbase = wid * WX
        pltpu.sync_copy(x_hbm.at[pl.ds(xbase, WX)], xv)

        jj = lax.iota(jnp.int32, 16)
        jb_lo = jj >> 3
        jb_hi = jb_lo + 2
        jr = jj & 7

        @plsc.parallel_loop(0, WX, 1, unroll=4)
        def _(r):
            rvec = jnp.full((16,), r, jnp.int32)
            for c in range(_SP // 16):
                plsc.store_scatter(
                    cv, [c * 16 + jj, rvec], xv[r, pl.ds(c * 16, 16)]
                )

        def fire_gather(s, bbl, slot):
            pltpu.async_copy(
                table_hbm.at[cv.at[s, pl.ds(bbl * _BB, _BB)]],
                rows_v.at[slot],
                gsems[slot],
            )

        def wait_gather(slot):
            pltpu.make_async_copy(
                table_hbm.at[cv.at[0, pl.ds(0, _BB)]],
                rows_v.at[slot],
                gsems[slot],
            ).wait()

        def scale_transpose(slot):
            @plsc.parallel_loop(0, _BB, 1, unroll=2)
            def _(r):
                rvec = jnp.full((16,), r, jnp.int32)
                v0 = rows_v[slot, r, pl.ds(0, 16)] * _SCALE
                plsc.store_scatter(tiles_v.at[slot], [jb_lo, jr, rvec], v0)
                v1 = rows_v[slot, r, pl.ds(16, 16)] * _SCALE
                plsc.store_scatter(tiles_v.at[slot], [jb_hi, jr, rvec], v1)

        def fire_stores(s, slot):
            bb = wid * BPW + slot
            for jb in range(_D // 8):
                pltpu.async_copy(
                    tiles_v.at[slot, jb], out_hbm.at[s, jb, bb], ssems[slot]
                )

        def wait_stores(slot):
            for jb in range(_D // 8):
                pltpu.make_async_copy(
                    tiles_v.at[slot, jb], out_hbm.at[0, 0, 0], ssems[slot]
                ).wait()

        for b in range(_NSLOT - 1):
            fire_gather(0, b, b)

        def outer(g2, _):
            for ph in range(_NSLOT):
                sp = (ph + _NSLOT - 1) % _NSLOT
                s_t = g2 if ph == 0 else g2 + 1

                if ph == 0:

                    @pl.when(g2 >= 1)
                    def _():
                        wait_stores(sp)

                    fire_gather(s_t, sp, sp)
                else:

                    @pl.when(g2 < S - 1)
                    def _():
                        wait_stores(sp)
                        fire_gather(s_t, sp, sp)

                wait_gather(ph)
                scale_transpose(ph)
                fire_stores(g2, ph)
            return 0

        lax.fori_loop(0, S, outer, 0)

        for b in range(_NSLOT):
            wait_stores(b)

    return k


@jax.jit
def kernel(x, weight):
    NB, S = x.shape
    xpad = jnp.pad(x.astype(jnp.int32), ((0, 0), (0, _SP - S)))
    out5 = _build(NB, S, weight.shape[0])(xpad, weight)
    return out5.transpose(2, 4, 0, 1, 3).reshape(NB, S, _D)


# speedup vs baseline: 3.1855x; 1.5373x over previous
import functools

import jax
import jax.numpy as jnp
from jax import lax
from jax.experimental import pallas as pl
from jax.experimental.pallas import tpu as pltpu
from jax.experimental.pallas import tpu_sc as plsc

_SCALE = 10.0
_D = 32
_BB = 128
_NSLOT = 4
_SP = 64


@functools.lru_cache(maxsize=None)
def _build(NB, S, V):
    info = plsc.get_sparse_core_info()
    NW = info.num_cores * info.num_subcores
    NBB = NB // _BB
    BPW = NBB // NW
    WX = NB // NW
    assert BPW == _NSLOT
    mesh = plsc.VectorSubcoreMesh(core_axis_name="c", subcore_axis_name="s")

    @functools.partial(
        pl.kernel,
        mesh=mesh,
        out_type=jax.ShapeDtypeStruct((S, _D // 8, NBB, 8, 128), jnp.float32),
        scratch_types=[
            pltpu.VMEM((WX, _SP), jnp.int32),
            pltpu.VMEM((_SP, WX + 1), jnp.int32),
            pltpu.VMEM((_NSLOT, _BB, _D), jnp.float32),
            pltpu.VMEM((_NSLOT, _D // 8, 8, 129), jnp.float32),
            [pltpu.SemaphoreType.DMA] * _NSLOT,
            [pltpu.SemaphoreType.DMA] * _NSLOT,
        ],
        compiler_params=pltpu.CompilerParams(
            use_tc_tiling_on_sc=False, needs_layout_passes=False
        ),
    )
    def k(x_hbm, table_hbm, out_hbm, xv, cv, rows_v, tiles_v, gsems, ssems):
        cid = lax.axis_index("c")
        sid = lax.axis_index("s")
        wid = sid * info.num_cores + cid
        xbase = wid * WX
        pltpu.sync_copy(x_hbm.at[pl.ds(xbase, WX)], xv)

        jj = lax.iota(jnp.int32, 16)
        jb_lo = jj >> 3
        jb_hi = jb_lo + 2
        jr = jj & 7

        @plsc.parallel_loop(0, WX, 1, unroll=4)
        def _(r):
            rvec = jnp.full((16,), r, jnp.int32)
            for c in range(_SP // 16):
                plsc.store_scatter(
                    cv, [c * 16 + jj, rvec], xv[r, pl.ds(c * 16, 16)]
                )

        def fire_gather(s, bbl, slot):
            pltpu.async_copy(
                table_hbm.at[cv.at[s, pl.ds(bbl * _BB, _BB)]],
                rows_v.at[slot],
                gsems[slot],
            )

        def wait_gather(slot):
            pltpu.make_async_copy(
                table_hbm.at[cv.at[0, pl.ds(0, _BB)]],
                rows_v.at[slot],
                gsems[slot],
            ).wait()

        def scale_transpose(slot):
            @plsc.parallel_loop(0, _BB, 1, unroll=2)
            def _(r):
                rvec = jnp.full((16,), r, jnp.int32)
                v0 = rows_v[slot, r, pl.ds(0, 16)] * _SCALE
                plsc.store_scatter(tiles_v.at[slot], [jb_lo, jr, rvec], v0)
                v1 = rows_v[slot, r, pl.ds(16, 16)] * _SCALE
                plsc.store_scatter(tiles_v.at[slot], [jb_hi, jr, rvec], v1)

        def fire_stores(s, slot):
            bb = wid * BPW + slot
            for jb in range(_D // 8):
                pltpu.async_copy(
                    tiles_v.at[slot, jb, :, pl.ds(0, 128)],
                    out_hbm.at[s, jb, bb],
                    ssems[slot],
                )

        def wait_stores(slot):
            for jb in range(_D // 8):
                pltpu.make_async_copy(
                    tiles_v.at[slot, jb, :, pl.ds(0, 128)],
                    out_hbm.at[0, 0, 0],
                    ssems[slot],
                ).wait()

        for b in range(_NSLOT - 1):
            fire_gather(0, b, b)

        def outer(g2, _):
            for ph in range(_NSLOT):
                sp = (ph + _NSLOT - 1) % _NSLOT
                s_t = g2 if ph == 0 else g2 + 1

                if ph == 0:

                    @pl.when(g2 >= 1)
                    def _():
                        wait_stores(sp)

                    fire_gather(s_t, sp, sp)
                else:

                    @pl.when(g2 < S - 1)
                    def _():
                        wait_stores(sp)
                        fire_gather(s_t, sp, sp)

                wait_gather(ph)
                scale_transpose(ph)
                fire_stores(g2, ph)
            return 0

        lax.fori_loop(0, S, outer, 0)

        for b in range(_NSLOT):
            wait_stores(b)

    return k


@jax.jit
def kernel(x, weight):
    NB, S = x.shape
    xpad = jnp.pad(x.astype(jnp.int32), ((0, 0), (0, _SP - S)))
    out5 = _build(NB, S, weight.shape[0])(xpad, weight)
    return out5.transpose(2, 4, 0, 1, 3).reshape(NB, S, _D)
